# baseline (speedup 1.0000x reference)
import jax
import jax.numpy as jnp
from jax import lax
from jax.experimental import pallas as pl
from jax.experimental.pallas import tpu as pltpu
from jax.experimental.pallas import tpu_sc as plsc

N = 10000
K = 16
CI = 16
CO = 16
NB = 8
L = 16
NC = 2
NS = 16
NW = NC * NS
NPAD = 10240
PPW = NPAD // NW
C = 8
NCH = PPW // C
SROW = NB * CI

GAMMA = (NB - 1) / 3.5
CENTERS = [3.5 * b / (NB - 1) for b in range(NB)]
GCENT = [GAMMA * c for c in CENTERS]

_BCAST_DNUMS = lax.GatherDimensionNumbers(
    offset_dims=(), collapsed_slice_dims=(0,), start_index_map=(0,))


def _bcast_lane(v, n):
    idx = jnp.full((L, 1), n, jnp.int32)
    return lax.gather(v, idx, _BCAST_DNUMS, slice_sizes=(1,),
                      mode=lax.GatherScatterMode.PROMISE_IN_BOUNDS)


def _sc_body(gx_h, gy_h, gz_h, nbr_h, msk_h, ftab_h, s_h,
             gx_v, gy_v, gz_v, nbr_v, msk_v, f_a, f_b, s_a, s_b,
             sga, sgb, ssa, ssb):
    wid = lax.axis_index("s") * NC + lax.axis_index("c")
    base0 = wid * PPW
    pltpu.async_copy(gx_h, gx_v, sga)
    pltpu.async_copy(gy_h, gy_v, sgb)
    pltpu.async_copy(gz_h, gz_v, ssa)
    pltpu.async_copy(nbr_h.at[pl.ds(wid * NCH, NCH)], nbr_v, ssb)
    pltpu.sync_copy(msk_h.at[pl.ds(base0 * K, PPW * K)], msk_v)
    pltpu.make_async_copy(gx_h, gx_v, sga).wait()
    pltpu.make_async_copy(gy_h, gy_v, sgb).wait()
    pltpu.make_async_copy(gz_h, gz_v, ssa).wait()
    pltpu.make_async_copy(nbr_h.at[pl.ds(wid * NCH, NCH)], nbr_v, ssb).wait()

    def process(c, f_v, s_v, cxa, cya, cza, lane_off):
        @plsc.parallel_loop(0, C, unroll=2)
        def _point(p):
            cx = _bcast_lane(cxa, lane_off + p)
            cy = _bcast_lane(cya, lane_off + p)
            cz = _bcast_lane(cza, lane_off + p)
            idx = nbr_v[c, pl.ds(p * K, K)]
            nx = plsc.load_gather(gx_v, [idx])
            ny = plsc.load_gather(gy_v, [idx])
            nz = plsc.load_gather(gz_v, [idx])
            dx = nx - cx
            dy = ny - cy
            dz = nz - cz
            d = dx * dx + dy * dy + dz * dz + 1e-12
            bits = lax.bitcast_convert_type(d, jnp.int32)
            bits = jnp.int32(0x5F3759DF) - lax.shift_right_logical(bits, 1)
            y = lax.bitcast_convert_type(bits, jnp.float32)
            for _ in range(3):
                y = y * (1.5 - 0.5 * d * y * y)
            r = d * y
            u = r * GAMMA
            mrow = msk_v[pl.ds((c * C + p) * K, K)]
            t4 = u - 4.0
            av = mrow * jnp.exp(-(t4 * t4))
            wv = jnp.exp(u + u)
            vv = jnp.exp(-(u + u))
            s0 = [jnp.zeros((L,), jnp.float32) for _ in range(NB)]
            s1 = [jnp.zeros((L,), jnp.float32) for _ in range(NB)]
            for n in range(K):
                frow = f_v[p * K + n, :]
                c4 = _bcast_lane(av, n)
                wb = _bcast_lane(wv, n)
                vb = _bcast_lane(vv, n)
                c5 = c4 * wb
                c6 = c5 * wb
                c7 = c6 * wb
                c3 = c4 * vb
                c2 = c3 * vb
                c1 = c2 * vb
                c0 = c1 * vb
                tgt = s0 if n % 2 == 0 else s1
                for b, cb in enumerate((c0, c1, c2, c3, c4, c5, c6, c7)):
                    tgt[b] = tgt[b] + cb * frow
            for b in range(NB):
                s_v[pl.ds(p * SROW + b * CI, L)] = s0[b] + s1[b]

    def store(c, s_v, sem):
        return pltpu.make_async_copy(
            s_v, s_h.at[pl.ds((base0 + c * C) * SROW, C * SROW)], sem)

    def gather(c, f_v, sem):
        return pltpu.make_async_copy(ftab_h.at[nbr_v.at[c]], f_v, sem)

    gather(0, f_a, sga).start()

    def pair(t, carry):
        c0 = 2 * t
        pb = base0 + c0 * C
        gather(c0 + 1, f_b, sgb).start()
        cxa = gx_v[pl.ds(pb, L)]
        cya = gy_v[pl.ds(pb, L)]
        cza = gz_v[pl.ds(pb, L)]
        gather(c0, f_a, sga).wait()

        @pl.when(t > 0)
        def _():
            store(c0 - 2, s_a, ssa).wait()
        process(c0, f_a, s_a, cxa, cya, cza, 0)
        store(c0, s_a, ssa).start()

        @pl.when(t + 1 < NCH // 2)
        def _():
            gather(c0 + 2, f_a, sga).start()
        gather(c0 + 1, f_b, sgb).wait()

        @pl.when(t > 0)
        def _():
            store(c0 - 1, s_b, ssb).wait()
        process(c0 + 1, f_b, s_b, cxa, cya, cza, C)
        store(c0 + 1, s_b, ssb).start()
        return carry

    lax.fori_loop(0, NCH // 2, pair, 0)
    store(NCH - 2, s_a, ssa).wait()
    store(NCH - 1, s_b, ssb).wait()


_sc_call = pl.kernel(
    _sc_body,
    out_type=jax.ShapeDtypeStruct((NPAD * SROW,), jnp.float32),
    mesh=plsc.VectorSubcoreMesh(core_axis_name="c", subcore_axis_name="s"),
    compiler_params=pltpu.CompilerParams(
        needs_layout_passes=False, use_tc_tiling_on_sc=False),
    scratch_types=[
        pltpu.VMEM((NPAD,), jnp.float32),
        pltpu.VMEM((NPAD,), jnp.float32),
        pltpu.VMEM((NPAD,), jnp.float32),
        pltpu.VMEM((NCH, C * K), jnp.int32),
        pltpu.VMEM((PPW * K,), jnp.float32),
        pltpu.VMEM((C * K, CI), jnp.float32),
        pltpu.VMEM((C * K, CI), jnp.float32),
        pltpu.VMEM((C * SROW,), jnp.float32),
        pltpu.VMEM((C * SROW,), jnp.float32),
        pltpu.SemaphoreType.DMA,
        pltpu.SemaphoreType.DMA,
        pltpu.SemaphoreType.DMA,
        pltpu.SemaphoreType.DMA,
    ],
)


def _mm_body(w_ref, s_ref, o_ref):
    o_ref[...] = lax.dot_general(
        w_ref[...], s_ref[...],
        dimension_numbers=(((1,), (1,)), ((), ())),
        preferred_element_type=jnp.float32)


def kernel(features, geometry, neighbors, rel_mask, W):
    geo = jnp.pad(geometry.astype(jnp.float32), ((0, NPAD - N), (0, 0)))
    gx = geo[:, 0]
    gy = geo[:, 1]
    gz = geo[:, 2]
    nbr = jnp.pad(neighbors.astype(jnp.int32), ((0, NPAD - N), (0, 0)))
    msk = jnp.pad(rel_mask.astype(jnp.float32), ((0, NPAD - N), (0, 0)))
    ftab = features.astype(jnp.float32).T
    s_flat = _sc_call(gx, gy, gz, nbr.reshape(-1, C * K), msk.reshape(-1),
                      ftab)
    S = s_flat.reshape(NPAD, SROW)
    kb = jnp.exp(jnp.float32(16.0) - jnp.arange(NB, dtype=jnp.float32) ** 2)
    W2 = (jnp.transpose(W.astype(jnp.float32), (0, 2, 1))
          * kb[None, :, None]).reshape(CO, SROW)
    out_pad = pl.pallas_call(
        _mm_body,
        out_shape=jax.ShapeDtypeStruct((CO, NPAD), jnp.float32),
    )(W2, S)
    return out_pad[:, :N]


# speedup vs baseline: 1.3853x; 1.3853x over previous
import jax
import jax.numpy as jnp
from jax import lax
from jax.experimental import pallas as pl
from jax.experimental.pallas import tpu as pltpu
from jax.experimental.pallas import tpu_sc as plsc

N = 10000
K = 16
CI = 16
CO = 16
NB = 8
L = 16
NC = 2
NS = 16
NW = NC * NS
NPAD = 10240
PPW = NPAD // NW
C = 8
NCH = PPW // C
SROW = NB * CI

GAMMA = (NB - 1) / 3.5
CENTERS = [3.5 * b / (NB - 1) for b in range(NB)]
GCENT = [GAMMA * c for c in CENTERS]

_BCAST_DNUMS = lax.GatherDimensionNumbers(
    offset_dims=(), collapsed_slice_dims=(0,), start_index_map=(0,))


def _bcast_lane(v, n):
    idx = jnp.full((L, 1), n, jnp.int32)
    return lax.gather(v, idx, _BCAST_DNUMS, slice_sizes=(1,),
                      mode=lax.GatherScatterMode.PROMISE_IN_BOUNDS)


def _sc_body(gx_h, gy_h, gz_h, nbr_h, msk_h, ftab_h, s_h,
             gx_v, gy_v, gz_v, nbr_v, msk_v, f_a, f_b, s_a, s_b,
             sga, sgb, ssa, ssb):
    wid = lax.axis_index("s") * NC + lax.axis_index("c")
    base0 = wid * PPW
    pltpu.async_copy(gx_h, gx_v, sga)
    pltpu.async_copy(gy_h, gy_v, sgb)
    pltpu.async_copy(gz_h, gz_v, ssa)
    pltpu.async_copy(nbr_h.at[pl.ds(wid * NCH, NCH)], nbr_v, ssb)
    pltpu.sync_copy(msk_h.at[pl.ds(base0 * K, PPW * K)], msk_v)
    pltpu.make_async_copy(gx_h, gx_v, sga).wait()
    pltpu.make_async_copy(gy_h, gy_v, sgb).wait()
    pltpu.make_async_copy(gz_h, gz_v, ssa).wait()
    pltpu.make_async_copy(nbr_h.at[pl.ds(wid * NCH, NCH)], nbr_v, ssb).wait()

    def process(c, f_v, s_v, cxa, cya, cza, lane_off):
        pts = []
        for p in range(C):
            cx = _bcast_lane(cxa, lane_off + p)
            cy = _bcast_lane(cya, lane_off + p)
            cz = _bcast_lane(cza, lane_off + p)
            idx = nbr_v[c, pl.ds(p * K, K)]
            nx = plsc.load_gather(gx_v, [idx])
            ny = plsc.load_gather(gy_v, [idx])
            nz = plsc.load_gather(gz_v, [idx])
            dx = nx - cx
            dy = ny - cy
            dz = nz - cz
            d = dx * dx + dy * dy + dz * dz + 1e-12
            bits = lax.bitcast_convert_type(d, jnp.int32)
            bits = jnp.int32(0x5F3759DF) - lax.shift_right_logical(bits, 1)
            y = lax.bitcast_convert_type(bits, jnp.float32)
            for _ in range(2):
                y = y * (1.5 - 0.5 * d * y * y)
            r = d * y
            u = r * GAMMA
            mrow = msk_v[pl.ds((c * C + p) * K, K)]
            t4 = u - 4.0
            av = mrow * jnp.exp(-(t4 * t4))
            wv = jnp.exp(u + u)
            vv = jnp.exp(-(u + u))
            pts.append((av, wv, vv))
        for p in range(C):
            av, wv, vv = pts[p]
            s0 = [jnp.zeros((L,), jnp.float32) for _ in range(NB)]
            s1 = [jnp.zeros((L,), jnp.float32) for _ in range(NB)]
            for n in range(K):
                frow = f_v[p * K + n, :]
                c4 = _bcast_lane(av, n)
                wb = _bcast_lane(wv, n)
                vb = _bcast_lane(vv, n)
                c5 = c4 * wb
                c6 = c5 * wb
                c7 = c6 * wb
                c3 = c4 * vb
                c2 = c3 * vb
                c1 = c2 * vb
                c0 = c1 * vb
                tgt = s0 if n % 2 == 0 else s1
                for b, cb in enumerate((c0, c1, c2, c3, c4, c5, c6, c7)):
                    tgt[b] = tgt[b] + cb * frow
            for b in range(NB):
                s_v[pl.ds(p * SROW + b * CI, L)] = s0[b] + s1[b]

    def store(c, s_v, sem):
        return pltpu.make_async_copy(
            s_v, s_h.at[pl.ds((base0 + c * C) * SROW, C * SROW)], sem)

    def gather(c, f_v, sem):
        return pltpu.make_async_copy(ftab_h.at[nbr_v.at[c]], f_v, sem)

    gather(0, f_a, sga).start()

    def pair(t, carry):
        c0 = 2 * t
        pb = base0 + c0 * C
        gather(c0 + 1, f_b, sgb).start()
        cxa = gx_v[pl.ds(pb, L)]
        cya = gy_v[pl.ds(pb, L)]
        cza = gz_v[pl.ds(pb, L)]
        gather(c0, f_a, sga).wait()

        @pl.when(t > 0)
        def _():
            store(c0 - 2, s_a, ssa).wait()
        process(c0, f_a, s_a, cxa, cya, cza, 0)
        store(c0, s_a, ssa).start()

        @pl.when(t + 1 < NCH // 2)
        def _():
            gather(c0 + 2, f_a, sga).start()
        gather(c0 + 1, f_b, sgb).wait()

        @pl.when(t > 0)
        def _():
            store(c0 - 1, s_b, ssb).wait()
        process(c0 + 1, f_b, s_b, cxa, cya, cza, C)
        store(c0 + 1, s_b, ssb).start()
        return carry

    lax.fori_loop(0, NCH // 2, pair, 0)
    store(NCH - 2, s_a, ssa).wait()
    store(NCH - 1, s_b, ssb).wait()


_sc_call = pl.kernel(
    _sc_body,
    out_type=jax.ShapeDtypeStruct((NPAD * SROW,), jnp.float32),
    mesh=plsc.VectorSubcoreMesh(core_axis_name="c", subcore_axis_name="s"),
    compiler_params=pltpu.CompilerParams(
        needs_layout_passes=False, use_tc_tiling_on_sc=False),
    scratch_types=[
        pltpu.VMEM((NPAD,), jnp.float32),
        pltpu.VMEM((NPAD,), jnp.float32),
        pltpu.VMEM((NPAD,), jnp.float32),
        pltpu.VMEM((NCH, C * K), jnp.int32),
        pltpu.VMEM((PPW * K,), jnp.float32),
        pltpu.VMEM((C * K, CI), jnp.float32),
        pltpu.VMEM((C * K, CI), jnp.float32),
        pltpu.VMEM((C * SROW,), jnp.float32),
        pltpu.VMEM((C * SROW,), jnp.float32),
        pltpu.SemaphoreType.DMA,
        pltpu.SemaphoreType.DMA,
        pltpu.SemaphoreType.DMA,
        pltpu.SemaphoreType.DMA,
    ],
)


def _mm_body(w_ref, s_ref, o_ref):
    full = lax.dot_general(
        w_ref[...], s_ref[...],
        dimension_numbers=(((1,), (1,)), ((), ())),
        preferred_element_type=jnp.float32)
    o_ref[...] = full[:, :N]


def kernel(features, geometry, neighbors, rel_mask, W):
    geo = jnp.pad(geometry.astype(jnp.float32), ((0, NPAD - N), (0, 0)))
    gx = geo[:, 0]
    gy = geo[:, 1]
    gz = geo[:, 2]
    nbr = jnp.pad(neighbors.astype(jnp.int32), ((0, NPAD - N), (0, 0)))
    msk = jnp.pad(rel_mask.astype(jnp.float32), ((0, NPAD - N), (0, 0)))
    ftab = features.astype(jnp.float32).T
    s_flat = _sc_call(gx, gy, gz, nbr.reshape(-1, C * K), msk.reshape(-1),
                      ftab)
    S = s_flat.reshape(NPAD, SROW)
    kb = jnp.exp(jnp.float32(16.0) - jnp.arange(NB, dtype=jnp.float32) ** 2)
    W2 = (jnp.transpose(W.astype(jnp.float32), (0, 2, 1))
          * kb[None, :, None]).reshape(CO, SROW)
    return pl.pallas_call(
        _mm_body,
        out_shape=jax.ShapeDtypeStruct((CO, N), jnp.float32),
    )(W2, S)


# speedup vs baseline: 1.3973x; 1.0087x over previous
import jax
import jax.numpy as jnp
from jax import lax
from jax.experimental import pallas as pl
from jax.experimental.pallas import tpu as pltpu
from jax.experimental.pallas import tpu_sc as plsc

N = 10000
K = 16
CI = 16
CO = 16
NB = 8
L = 16
NC = 2
NS = 16
NW = NC * NS
NPAD = 10240
PPW = NPAD // NW
C = 8
NCH = PPW // C
SROW = NB * CI

GAMMA = (NB - 1) / 3.5
CENTERS = [3.5 * b / (NB - 1) for b in range(NB)]
GCENT = [GAMMA * c for c in CENTERS]

_BCAST_DNUMS = lax.GatherDimensionNumbers(
    offset_dims=(), collapsed_slice_dims=(0,), start_index_map=(0,))


def _bcast_lane(v, n):
    idx = jnp.full((L, 1), n, jnp.int32)
    return lax.gather(v, idx, _BCAST_DNUMS, slice_sizes=(1,),
                      mode=lax.GatherScatterMode.PROMISE_IN_BOUNDS)


def _sc_body(gx_h, gy_h, gz_h, nbr_h, msk_h, ftab_h, s_h,
             gx_v, gy_v, gz_v, nbr_v, msk_v, f_a, f_b, s_a, s_b,
             sga, sgb, ssa, ssb):
    wid = lax.axis_index("s") * NC + lax.axis_index("c")
    base0 = wid * PPW
    pltpu.async_copy(gx_h, gx_v, sga)
    pltpu.async_copy(gy_h, gy_v, sgb)
    pltpu.async_copy(gz_h, gz_v, ssa)
    pltpu.async_copy(nbr_h.at[pl.ds(wid * NCH, NCH)], nbr_v, ssb)
    pltpu.sync_copy(msk_h.at[pl.ds(base0 * K, PPW * K)], msk_v)
    pltpu.make_async_copy(gx_h, gx_v, sga).wait()
    pltpu.make_async_copy(gy_h, gy_v, sgb).wait()
    pltpu.make_async_copy(gz_h, gz_v, ssa).wait()
    pltpu.make_async_copy(nbr_h.at[pl.ds(wid * NCH, NCH)], nbr_v, ssb).wait()

    def process(c, f_v, s_v, cxa, cya, cza, lane_off):
        pts = []
        for p in range(C):
            cx = _bcast_lane(cxa, lane_off + p)
            cy = _bcast_lane(cya, lane_off + p)
            cz = _bcast_lane(cza, lane_off + p)
            idx = nbr_v[c, pl.ds(p * K, K)]
            nx = plsc.load_gather(gx_v, [idx])
            ny = plsc.load_gather(gy_v, [idx])
            nz = plsc.load_gather(gz_v, [idx])
            dx = nx - cx
            dy = ny - cy
            dz = nz - cz
            d = dx * dx + dy * dy + dz * dz + 1e-12
            bits = lax.bitcast_convert_type(d, jnp.int32)
            bits = jnp.int32(0x5F3759DF) - lax.shift_right_logical(bits, 1)
            y = lax.bitcast_convert_type(bits, jnp.float32)
            for _ in range(2):
                y = y * (1.5 - 0.5 * d * y * y)
            r = d * y
            u = r * GAMMA
            mrow = msk_v[pl.ds((c * C + p) * K, K)]
            t4 = u - 4.0
            av = mrow * jnp.exp(-(t4 * t4))
            wv = jnp.exp(u + u)
            vv = jnp.exp(-(u + u))
            pts.append((av, wv, vv))
        for p in range(C):
            av, wv, vv = pts[p]
            s0 = [jnp.zeros((L,), jnp.float32) for _ in range(NB)]
            s1 = [jnp.zeros((L,), jnp.float32) for _ in range(NB)]
            for n in range(K):
                frow = f_v[p * K + n, :]
                c4 = _bcast_lane(av, n)
                wb = _bcast_lane(wv, n)
                vb = _bcast_lane(vv, n)
                c5 = c4 * wb
                c6 = c5 * wb
                c7 = c6 * wb
                c3 = c4 * vb
                c2 = c3 * vb
                c1 = c2 * vb
                c0 = c1 * vb
                tgt = s0 if n % 2 == 0 else s1
                for b, cb in enumerate((c0, c1, c2, c3, c4, c5, c6, c7)):
                    tgt[b] = tgt[b] + cb * frow
            for b in range(NB):
                s_v[pl.ds(p * SROW + b * CI, L)] = s0[b] + s1[b]

    def store(c, s_v, sem):
        return pltpu.make_async_copy(
            s_v, s_h.at[pl.ds((base0 + c * C) * SROW, C * SROW)], sem)

    def gather(c, f_v, sem):
        return pltpu.make_async_copy(ftab_h.at[nbr_v.at[c]], f_v, sem)

    gather(0, f_a, sga).start()

    def pair(t, carry):
        c0 = 2 * t
        pb = base0 + c0 * C
        gather(c0 + 1, f_b, sgb).start()
        cxa = gx_v[pl.ds(pb, L)]
        cya = gy_v[pl.ds(pb, L)]
        cza = gz_v[pl.ds(pb, L)]
        gather(c0, f_a, sga).wait()

        @pl.when(t > 0)
        def _():
            store(c0 - 2, s_a, ssa).wait()
        process(c0, f_a, s_a, cxa, cya, cza, 0)
        store(c0, s_a, ssa).start()

        @pl.when(t + 1 < NCH // 2)
        def _():
            gather(c0 + 2, f_a, sga).start()
        gather(c0 + 1, f_b, sgb).wait()

        @pl.when(t > 0)
        def _():
            store(c0 - 1, s_b, ssb).wait()
        process(c0 + 1, f_b, s_b, cxa, cya, cza, C)
        store(c0 + 1, s_b, ssb).start()
        return carry

    lax.fori_loop(0, NCH // 2, pair, 0)
    store(NCH - 2, s_a, ssa).wait()
    store(NCH - 1, s_b, ssb).wait()


_sc_call = pl.kernel(
    _sc_body,
    out_type=jax.ShapeDtypeStruct((NPAD * SROW,), jnp.float32),
    mesh=plsc.VectorSubcoreMesh(core_axis_name="c", subcore_axis_name="s"),
    compiler_params=pltpu.CompilerParams(
        needs_layout_passes=False, use_tc_tiling_on_sc=False,
        disable_bounds_checks=True),
    scratch_types=[
        pltpu.VMEM((NPAD,), jnp.float32),
        pltpu.VMEM((NPAD,), jnp.float32),
        pltpu.VMEM((NPAD,), jnp.float32),
        pltpu.VMEM((NCH, C * K), jnp.int32),
        pltpu.VMEM((PPW * K,), jnp.float32),
        pltpu.VMEM((C * K, CI), jnp.float32),
        pltpu.VMEM((C * K, CI), jnp.float32),
        pltpu.VMEM((C * SROW,), jnp.float32),
        pltpu.VMEM((C * SROW,), jnp.float32),
        pltpu.SemaphoreType.DMA,
        pltpu.SemaphoreType.DMA,
        pltpu.SemaphoreType.DMA,
        pltpu.SemaphoreType.DMA,
    ],
)


def _mm_body(w_ref, s_ref, o_ref):
    full = lax.dot_general(
        w_ref[...], s_ref[...],
        dimension_numbers=(((1,), (1,)), ((), ())),
        preferred_element_type=jnp.float32)
    o_ref[...] = full[:, :N]


def kernel(features, geometry, neighbors, rel_mask, W):
    geo = jnp.pad(geometry.astype(jnp.float32), ((0, NPAD - N), (0, 0)))
    gx = geo[:, 0]
    gy = geo[:, 1]
    gz = geo[:, 2]
    nbr = jnp.pad(neighbors.astype(jnp.int32), ((0, NPAD - N), (0, 0)))
    msk = jnp.pad(rel_mask.astype(jnp.float32), ((0, NPAD - N), (0, 0)))
    ftab = features.astype(jnp.float32).T
    s_flat = _sc_call(gx, gy, gz, nbr.reshape(-1, C * K), msk.reshape(-1),
                      ftab)
    S = s_flat.reshape(NPAD, SROW)
    kb = jnp.exp(jnp.float32(16.0) - jnp.arange(NB, dtype=jnp.float32) ** 2)
    W2 = (jnp.transpose(W.astype(jnp.float32), (0, 2, 1))
          * kb[None, :, None]).reshape(CO, SROW)
    return pl.pallas_call(
        _mm_body,
        out_shape=jax.ShapeDtypeStruct((CO, N), jnp.float32),
    )(W2, S)
